# baseline (speedup 1.0000x reference)
import jax
import jax.numpy as jnp
from jax.experimental import pallas as pl
from jax.experimental.pallas import tpu as pltpu


def kernel(tokens, W_pos):
    B = tokens.shape[0]
    S = tokens.shape[1]
    D = W_pos.shape[1]

    def body(w_ref, out_hbm, out_sem):
        copies = [
            pltpu.async_copy(w_ref, out_hbm.at[b], out_sem) for b in range(B)
        ]
        for c in copies:
            c.wait()

    return pl.pallas_call(
        body,
        in_specs=[pl.BlockSpec(memory_space=pltpu.MemorySpace.VMEM)],
        out_specs=pl.BlockSpec(memory_space=pltpu.MemorySpace.HBM),
        out_shape=jax.ShapeDtypeStruct((B, S, D), W_pos.dtype),
        scratch_shapes=[
            pltpu.SemaphoreType.DMA,
        ],
    )(W_pos[:S])
